# baseline (speedup 1.0000x reference)
import functools

import jax
import jax.numpy as jnp
from jax import lax
from jax.experimental import pallas as pl
from jax.experimental.pallas import tpu as pltpu
from jax.experimental.pallas import tpu_sc as plsc

VOCAB = 1000000
EMBED_DIM = 64
SEQ_LEN = 200
BATCH = 4096

NUM_CORES = 2
NUM_SUBCORES = 16
LANES = 16
NUM_WORKERS = NUM_CORES * NUM_SUBCORES

TOTAL = BATCH * SEQ_LEN
PER_WORKER = TOTAL // NUM_WORKERS
CHUNK = 512
SUB = 128
K = CHUNK // SUB
SUPER = 2 * CHUNK
NUM_SUPER = PER_WORKER // SUPER


def _sc_kernel(idx_hbm, table_hbm, pos_hbm, out_hbm, idx_v, rows_v, pos_v,
               gsem):
  wid = lax.axis_index("s") * NUM_CORES + lax.axis_index("c")
  base = wid * PER_WORKER

  pltpu.sync_copy(pos_hbm, pos_v)

  def super_body(si, carry):
    soff = base + si * SUPER
    idx_row0 = pl.multiple_of(soff // SUB, 8)
    pltpu.sync_copy(idx_hbm.at[pl.ds(idx_row0, SUPER // SUB)], idx_v)
    for h in range(SUPER // CHUNK):
      off = soff + h * CHUNK
      copies = []
      for j in range(K):
        copies.append(
            pltpu.async_copy(table_hbm.at[idx_v.at[h * K + j]],
                             rows_v.at[pl.ds(j * SUB, SUB)], gsem))
      for c in copies:
        c.wait()

      def row_body(r, carry2):
        pr = lax.rem(off + r, SEQ_LEN)
        for cblk in range(EMBED_DIM // LANES):
          sl = pl.ds(cblk * LANES, LANES)
          rows_v[r, sl] = rows_v[r, sl] + pos_v[pr, sl]
        return carry2

      lax.fori_loop(0, CHUNK, row_body, 0)
      pltpu.sync_copy(rows_v, out_hbm.at[pl.ds(off, CHUNK)])
    return carry

  lax.fori_loop(0, NUM_SUPER, super_body, 0)


@jax.jit
def _run(idx2d, word_table, pos_table):
  mesh = plsc.VectorSubcoreMesh(core_axis_name="c", subcore_axis_name="s")
  f = functools.partial(
      pl.kernel,
      mesh=mesh,
      compiler_params=pltpu.CompilerParams(use_tc_tiling_on_sc=False),
      out_type=jax.ShapeDtypeStruct((TOTAL, EMBED_DIM), jnp.float32),
      scratch_types=[
          pltpu.VMEM((SUPER // SUB, SUB), jnp.int32),
          pltpu.VMEM((CHUNK, EMBED_DIM), jnp.float32),
          pltpu.VMEM((SEQ_LEN, EMBED_DIM), jnp.float32),
          pltpu.SemaphoreType.DMA,
      ],
  )(_sc_kernel)
  return f(idx2d, word_table, pos_table)


def kernel(input_idx, word_table, pos_table):
  idx2d = input_idx.astype(jnp.int32).reshape(TOTAL // SUB, SUB)
  out = _run(idx2d, word_table, pos_table)
  return out.reshape(BATCH, SEQ_LEN, EMBED_DIM)


# speedup vs baseline: 1.2441x; 1.2441x over previous
import functools

import jax
import jax.numpy as jnp
from jax import lax
from jax.experimental import pallas as pl
from jax.experimental.pallas import tpu as pltpu
from jax.experimental.pallas import tpu_sc as plsc

VOCAB = 1000000
EMBED_DIM = 64
SEQ_LEN = 200
BATCH = 4096

NUM_CORES = 2
NUM_SUBCORES = 16
LANES = 16
NUM_WORKERS = NUM_CORES * NUM_SUBCORES

TOTAL = BATCH * SEQ_LEN
PER_WORKER = TOTAL // NUM_WORKERS
CHUNK = 512
SUB = 128
K = CHUNK // SUB
SUPER = 2 * CHUNK
IDXROWS = SUPER // SUB
NUM_SUPER = PER_WORKER // SUPER
NUM_CHUNKS = PER_WORKER // CHUNK
POS4 = 4 * SEQ_LEN
ROWS_PER_ITER = 4


def _sc_kernel(idx_hbm, table_hbm, pos_hbm, out_hbm, idx0_v, idx1_v,
               rows0_v, rows1_v, pos4_v, gsem):
  wid = lax.axis_index("s") * NUM_CORES + lax.axis_index("c")
  base = wid * PER_WORKER
  idx_base = wid * (PER_WORKER // SUB)

  idx_bufs = (idx0_v, idx1_v)
  row_bufs = (rows0_v, rows1_v)

  for rep in range(POS4 // SEQ_LEN):
    pltpu.sync_copy(pos_hbm, pos4_v.at[pl.ds(rep * SEQ_LEN, SEQ_LEN)])

  def load_idx(s):
    row0 = pl.multiple_of(idx_base + s * IDXROWS, 8)
    pltpu.sync_copy(idx_hbm.at[pl.ds(row0, IDXROWS)], idx_bufs[s % 2])

  def fire_gathers(ci):
    s, half, p = ci // 2, ci % 2, ci % 2
    return [
        pltpu.async_copy(table_hbm.at[idx_bufs[s % 2].at[half * K + j]],
                         row_bufs[p].at[pl.ds(j * SUB, SUB)], gsem)
        for j in range(K)
    ]

  def add_pos(ci):
    buf = row_bufs[ci % 2]
    p0 = (ci * CHUNK) % SEQ_LEN

    def body(r0, carry):
      for rr in range(ROWS_PER_ITER):
        row = r0 * ROWS_PER_ITER + rr
        for cb in range(EMBED_DIM // LANES):
          sl = pl.ds(cb * LANES, LANES)
          buf[row, sl] = buf[row, sl] + pos4_v[p0 + row, sl]
      return carry

    lax.fori_loop(0, CHUNK // ROWS_PER_ITER, body, 0)

  def store(ci):
    off = base + ci * CHUNK
    pltpu.sync_copy(row_bufs[ci % 2], out_hbm.at[pl.ds(off, CHUNK)])

  for ci in range(NUM_CHUNKS):
    s = ci // 2
    if ci % 2 == 0:
      load_idx(s)
    for d in fire_gathers(ci):
      d.wait()
    add_pos(ci)
    store(ci)


@jax.jit
def _run(idx2d, word_table, pos_table):
  mesh = plsc.VectorSubcoreMesh(core_axis_name="c", subcore_axis_name="s")
  f = functools.partial(
      pl.kernel,
      mesh=mesh,
      compiler_params=pltpu.CompilerParams(use_tc_tiling_on_sc=False),
      out_type=jax.ShapeDtypeStruct((TOTAL, EMBED_DIM), jnp.float32),
      scratch_types=[
          pltpu.VMEM((IDXROWS, SUB), jnp.int32),
          pltpu.VMEM((IDXROWS, SUB), jnp.int32),
          pltpu.VMEM((CHUNK, EMBED_DIM), jnp.float32),
          pltpu.VMEM((CHUNK, EMBED_DIM), jnp.float32),
          pltpu.VMEM((POS4, EMBED_DIM), jnp.float32),
          pltpu.SemaphoreType.DMA,
      ],
  )(_sc_kernel)
  return f(idx2d, word_table, pos_table)


def kernel(input_idx, word_table, pos_table):
  idx2d = input_idx.astype(jnp.int32).reshape(TOTAL // SUB, SUB)
  out = _run(idx2d, word_table, pos_table)
  return out.reshape(BATCH, SEQ_LEN, EMBED_DIM)
